# baseline (speedup 1.0000x reference)
import functools

import jax
import jax.numpy as jnp
from jax import lax
from jax.experimental import pallas as pl
from jax.experimental.pallas import tpu as pltpu
from jax.experimental.pallas import tpu_sc as plsc

_N_NODES = 10000
_N_EDGES = 320000
_D = 128
_D_EDGE = 16
_NUM_GRAPHS = 64
_NUM_CLASS = 10

_NC = 2
_NS = 16
_NW = _NC * _NS
_EPW = _N_EDGES // _NW
_CG = 200
_CS = 40
_NPAD = 10240
_ZR = 80
_RPT = _NPAD // _NS


_MASK_HI = -65536
_HALF = 0x8000


def _pack2(hi_f32, lo_f32):
    bh = lax.bitcast_convert_type(hi_f32, jnp.int32)
    bl = lax.bitcast_convert_type(lo_f32, jnp.int32)
    h = (bh + _HALF) & _MASK_HI
    l = lax.shift_right_logical(bl + _HALF, 16)
    return h | l


def _unpack_hi(p_i32):
    return lax.bitcast_convert_type(p_i32 & _MASK_HI, jnp.float32)


def _unpack_lo(p_i32):
    return lax.bitcast_convert_type(lax.shift_left(p_i32, 16), jnp.float32)


def _pq_body(x_ref, wd_ref, ws_ref, p_ref, q_ref):
    xb = x_ref[...]
    pf = jnp.dot(xb, wd_ref[...], preferred_element_type=jnp.float32)
    qf = jnp.dot(xb, ws_ref[...], preferred_element_type=jnp.float32)
    p_ref[...] = _pack2(pf[:, :_D], pf[:, _D:])
    q_ref[...] = _pack2(qf[:, :_D], qf[:, _D:])


def _edge_body(pg_ref, qg_ref, ea_ref, we_ref, b_ref, out_ref):
    pg = pg_ref[...]
    qg = qg_ref[...]
    e = (jnp.dot(ea_ref[...], we_ref[...], preferred_element_type=jnp.float32)
         + b_ref[...])
    tf = _unpack_hi(pg) + _unpack_hi(qg) + e[:, :_D]
    ts = _unpack_lo(pg) + _unpack_lo(qg) + e[:, _D:]
    gate = 1.0 / (1.0 + jnp.exp(-tf))
    core = jnp.maximum(ts, 0.0) + jnp.log(1.0 + jnp.exp(-jnp.abs(ts)))
    out_ref[...] = gate * core


def _final_body(p_ref, x_ref, batch_ref, wl_ref, bl_ref, out_ref):
    agg = p_ref[0, :_N_NODES, :] + p_ref[1, :_N_NODES, :]
    out = jnp.maximum(agg + x_ref[...], 0.0)
    gids = lax.broadcasted_iota(jnp.int32, (_NUM_GRAPHS, _N_NODES), 0)
    m = (batch_ref[...] == gids).astype(jnp.float32)
    sums = jnp.dot(m, out, preferred_element_type=jnp.float32)
    counts = jnp.sum(m, axis=1, keepdims=True)
    pooled = sums / jnp.maximum(counts, 1.0)
    out_ref[...] = (jnp.dot(pooled, wl_ref[...],
                            preferred_element_type=jnp.float32) + bl_ref[...])


def _tc_pq(x, wd, ws):
    nb = 5
    rows = _N_NODES // nb
    return pl.pallas_call(
        _pq_body,
        grid=(nb,),
        in_specs=[
            pl.BlockSpec((rows, _D), lambda i: (i, 0)),
            pl.BlockSpec((_D, 2 * _D), lambda i: (0, 0)),
            pl.BlockSpec((_D, 2 * _D), lambda i: (0, 0)),
        ],
        out_specs=[
            pl.BlockSpec((rows, _D), lambda i: (i, 0)),
            pl.BlockSpec((rows, _D), lambda i: (i, 0)),
        ],
        out_shape=[jax.ShapeDtypeStruct((_N_NODES, _D), jnp.int32)] * 2,
    )(x, wd, ws)


def _tc_edge(pg, qg, ea, we, bcat):
    be = 2000
    nb = _N_EDGES // be
    return pl.pallas_call(
        _edge_body,
        grid=(nb,),
        in_specs=[
            pl.BlockSpec((be, _D), lambda i: (i, 0)),
            pl.BlockSpec((be, _D), lambda i: (i, 0)),
            pl.BlockSpec((be, _D_EDGE), lambda i: (i, 0)),
            pl.BlockSpec((_D_EDGE, 2 * _D), lambda i: (0, 0)),
            pl.BlockSpec((1, 2 * _D), lambda i: (0, 0)),
        ],
        out_specs=pl.BlockSpec((be, _D), lambda i: (i, 0)),
        out_shape=jax.ShapeDtypeStruct((_N_EDGES, _D), jnp.float32),
    )(pg, qg, ea, we, bcat)


def _tc_final(partials, x, batch2d, wlt, bl2d):
    return pl.pallas_call(
        _final_body,
        out_shape=jax.ShapeDtypeStruct((_NUM_GRAPHS, _NUM_CLASS), jnp.float32),
    )(partials, x, batch2d, wlt, bl2d)


_NCHUNK = _EPW // _CG


def _sc_gather(p, q, dst, src):
    mesh = plsc.VectorSubcoreMesh(core_axis_name="c", subcore_axis_name="s")

    @functools.partial(
        pl.kernel,
        mesh=mesh,
        out_type=[jax.ShapeDtypeStruct((_N_EDGES, _D), jnp.int32),
                  jax.ShapeDtypeStruct((_N_EDGES, _D), jnp.int32)],
        scratch_types=[
            pltpu.VMEM((_CG,), jnp.int32),
            pltpu.VMEM((_CG,), jnp.int32),
            pltpu.VMEM((_CG,), jnp.int32),
            pltpu.VMEM((_CG,), jnp.int32),
            pltpu.VMEM((2, _CG, _D), jnp.int32),
            pltpu.VMEM((2, _CG, _D), jnp.int32),
        ] + [pltpu.SemaphoreType.DMA] * 4,
    )
    def k(p_hbm, q_hbm, dst_hbm, src_hbm, pg_hbm, qg_hbm,
          idxd0, idxd1, idxs0, idxs1, bufp, bufq, gsem0, gsem1, wsem0, wsem1):
        wid = lax.axis_index("s") * _NC + lax.axis_index("c")
        idxd = (idxd0, idxd1)
        idxs = (idxs0, idxs1)
        gsems = (gsem0, gsem1)
        wsems = (wsem0, wsem1)

        def start(i, b):
            base = wid * _EPW + i * _CG
            pltpu.sync_copy(dst_hbm.at[pl.ds(base, _CG)], idxd[b])
            pltpu.sync_copy(src_hbm.at[pl.ds(base, _CG)], idxs[b])
            pltpu.async_copy(p_hbm.at[idxd[b]], bufp.at[b], gsems[b])
            pltpu.async_copy(q_hbm.at[idxs[b]], bufq.at[b], gsems[b])

        def wait_gather(b):
            pltpu.make_async_copy(p_hbm.at[idxd[b]], bufp.at[b], gsems[b]).wait()
            pltpu.make_async_copy(q_hbm.at[idxs[b]], bufq.at[b], gsems[b]).wait()

        def put(i, b):
            base = wid * _EPW + i * _CG
            pltpu.async_copy(bufp.at[b], pg_hbm.at[pl.ds(base, _CG)], wsems[b])
            pltpu.async_copy(bufq.at[b], qg_hbm.at[pl.ds(base, _CG)], wsems[b])

        def wait_put(i, b):
            base = wid * _EPW + i * _CG
            pltpu.make_async_copy(bufp.at[b], pg_hbm.at[pl.ds(base, _CG)], wsems[b]).wait()
            pltpu.make_async_copy(bufq.at[b], qg_hbm.at[pl.ds(base, _CG)], wsems[b]).wait()

        start(0, 0)
        start(1, 1)

        def body(j, carry):
            for b in range(2):
                i = 2 * j + b
                wait_gather(b)
                put(i, b)
                wait_put(i, b)
                start(i + 2, b)
            return carry

        lax.fori_loop(0, _NCHUNK // 2 - 1, body, 0)
        for b in range(2):
            i = _NCHUNK - 2 + b
            wait_gather(b)
            put(i, b)
            wait_put(i, b)

    return k(p, q, dst, src)


def _sc_scatter(msg, dst):
    mesh = plsc.VectorSubcoreMesh(core_axis_name="c", subcore_axis_name="s")

    @functools.partial(
        pl.kernel,
        mesh=mesh,
        out_type=jax.ShapeDtypeStruct((_NC, _NPAD, _D), jnp.float32),
        scratch_types=[
            pltpu.VMEM((_CS,), jnp.int32),
            pltpu.VMEM((_CS,), jnp.int32),
            pltpu.VMEM((2, _CS, _D), jnp.float32),
            pltpu.VMEM((_ZR, _D), jnp.float32),
            pltpu.VMEM_SHARED((_NPAD, _D), jnp.float32),
        ] + [pltpu.SemaphoreType.DMA] * 2,
    )
    def k(msg_hbm, dst_hbm, out_hbm, idx0, idx1, buf, zbuf, acc, lsem0, lsem1):
        cid = lax.axis_index("c")
        sid = lax.axis_index("s")
        wid = sid * _NC + cid
        idx = (idx0, idx1)
        lsems = (lsem0, lsem1)

        def zrow(r, carry):
            for kk in range(_D // 16):
                zbuf[r, pl.ds(kk * 16, 16)] = jnp.zeros((16,), jnp.float32)
            return carry

        lax.fori_loop(0, _ZR, zrow, 0)

        def zcopy(j, carry):
            pltpu.sync_copy(zbuf, acc.at[pl.ds(sid * _RPT + j * _ZR, _ZR)])
            return carry

        lax.fori_loop(0, _RPT // _ZR, zcopy, 0)
        plsc.subcore_barrier()

        def start(i, b):
            base = wid * _EPW + i * _CS
            pltpu.async_copy(dst_hbm.at[pl.ds(base, _CS)], idx[b], lsems[b])
            pltpu.async_copy(msg_hbm.at[pl.ds(base, _CS)], buf.at[b], lsems[b])

        def wait_load(i, b):
            base = wid * _EPW + i * _CS
            pltpu.make_async_copy(dst_hbm.at[pl.ds(base, _CS)], idx[b],
                                  lsems[b]).wait()
            pltpu.make_async_copy(msg_hbm.at[pl.ds(base, _CS)], buf.at[b],
                                  lsems[b]).wait()

        nchunk = _EPW // _CS
        start(0, 0)
        start(1, 1)

        def body(j, carry):
            for b in range(2):
                i = 2 * j + b
                wait_load(i, b)
                pltpu.sync_copy(buf.at[b], acc.at[idx[b]], add=True)
                start(i + 2, b)
            return carry

        lax.fori_loop(0, nchunk // 2 - 1, body, 0)
        for b in range(2):
            i = nchunk - 2 + b
            wait_load(i, b)
            pltpu.sync_copy(buf.at[b], acc.at[idx[b]], add=True)
        plsc.subcore_barrier()

        def drain(j, carry):
            r0 = sid * _RPT + j * _ZR
            pltpu.sync_copy(acc.at[pl.ds(r0, _ZR)], zbuf)
            pltpu.sync_copy(zbuf, out_hbm.at[cid, pl.ds(r0, _ZR)])
            return carry

        lax.fori_loop(0, _RPT // _ZR, drain, 0)

    return k(msg, dst)


def kernel(x, edge_index, edge_attr, batch, W_f, b_f, W_s, b_s, W_lin, b_lin):
    src = edge_index[0].astype(jnp.int32)
    dst = edge_index[1].astype(jnp.int32)

    wd = jnp.concatenate([W_f[:, :_D], W_s[:, :_D]], axis=0).T
    ws = jnp.concatenate([W_f[:, _D:2 * _D], W_s[:, _D:2 * _D]], axis=0).T
    we = jnp.concatenate([W_f[:, 2 * _D:], W_s[:, 2 * _D:]], axis=0).T
    bcat = jnp.concatenate([b_f, b_s]).reshape(1, 2 * _D)
    wlt = W_lin.T
    bl2d = b_lin.reshape(1, _NUM_CLASS)
    batch2d = batch.astype(jnp.int32).reshape(1, _N_NODES)

    p, q = _tc_pq(x, wd, ws)
    pg, qg = _sc_gather(p, q, dst, src)
    msg = _tc_edge(pg, qg, edge_attr, we, bcat)
    partials = _sc_scatter(msg, dst)
    return _tc_final(partials, x, batch2d, wlt, bl2d)


# speedup vs baseline: 1.0858x; 1.0858x over previous
import functools

import jax
import jax.numpy as jnp
from jax import lax
from jax.experimental import pallas as pl
from jax.experimental.pallas import tpu as pltpu
from jax.experimental.pallas import tpu_sc as plsc

_N_NODES = 10000
_N_EDGES = 320000
_D = 128
_D_EDGE = 16
_NUM_GRAPHS = 64
_NUM_CLASS = 10

_NC = 2
_NS = 16
_NW = _NC * _NS
_EPW = _N_EDGES // _NW
_CG = 200
_CS = 40
_NPAD = 10240
_ZR = 80
_RPT = _NPAD // _NS


_MASK_HI = -65536
_HALF = 0x8000


def _pack2(hi_f32, lo_f32):
    bh = lax.bitcast_convert_type(hi_f32, jnp.int32)
    bl = lax.bitcast_convert_type(lo_f32, jnp.int32)
    h = (bh + _HALF) & _MASK_HI
    l = lax.shift_right_logical(bl + _HALF, 16)
    return h | l


def _unpack_hi(p_i32):
    return lax.bitcast_convert_type(p_i32 & _MASK_HI, jnp.float32)


def _unpack_lo(p_i32):
    return lax.bitcast_convert_type(lax.shift_left(p_i32, 16), jnp.float32)


def _pq_body(x_ref, wd_ref, ws_ref, p_ref, q_ref):
    xb = x_ref[...]
    pf = jnp.dot(xb, wd_ref[...], preferred_element_type=jnp.float32)
    qf = jnp.dot(xb, ws_ref[...], preferred_element_type=jnp.float32)
    p_ref[...] = _pack2(pf[:, :_D], pf[:, _D:])
    q_ref[...] = _pack2(qf[:, :_D], qf[:, _D:])


def _edge_body(pg_ref, qg_ref, ea_ref, we_ref, b_ref, out_ref):
    pg = pg_ref[...]
    qg = qg_ref[...]
    e = (jnp.dot(ea_ref[...], we_ref[...], preferred_element_type=jnp.float32)
         + b_ref[...])
    tf = _unpack_hi(pg) + _unpack_hi(qg) + e[:, :_D]
    ts = _unpack_lo(pg) + _unpack_lo(qg) + e[:, _D:]
    gate = 1.0 / (1.0 + jnp.exp(-tf))
    core = jnp.maximum(ts, 0.0) + jnp.log(1.0 + jnp.exp(-jnp.abs(ts)))
    out_ref[...] = gate * core


def _final_body(pa_ref, pb_ref, x_ref, batch_ref, wl_ref, bl_ref, out_ref):
    agg = (pa_ref[0, :_N_NODES, :] + pa_ref[1, :_N_NODES, :]
           + pb_ref[0, :_N_NODES, :] + pb_ref[1, :_N_NODES, :])
    out = jnp.maximum(agg + x_ref[...], 0.0)
    gids = lax.broadcasted_iota(jnp.int32, (_NUM_GRAPHS, _N_NODES), 0)
    m = (batch_ref[...] == gids).astype(jnp.float32)
    sums = jnp.dot(m, out, preferred_element_type=jnp.float32)
    counts = jnp.sum(m, axis=1, keepdims=True)
    pooled = sums / jnp.maximum(counts, 1.0)
    out_ref[...] = (jnp.dot(pooled, wl_ref[...],
                            preferred_element_type=jnp.float32) + bl_ref[...])


def _tc_pq(x, wd, ws):
    nb = 5
    rows = _N_NODES // nb
    return pl.pallas_call(
        _pq_body,
        grid=(nb,),
        in_specs=[
            pl.BlockSpec((rows, _D), lambda i: (i, 0)),
            pl.BlockSpec((_D, 2 * _D), lambda i: (0, 0)),
            pl.BlockSpec((_D, 2 * _D), lambda i: (0, 0)),
        ],
        out_specs=[
            pl.BlockSpec((rows, _D), lambda i: (i, 0)),
            pl.BlockSpec((rows, _D), lambda i: (i, 0)),
        ],
        out_shape=[jax.ShapeDtypeStruct((_N_NODES, _D), jnp.int32)] * 2,
    )(x, wd, ws)


def _tc_edge(pg, qg, ea, we, bcat):
    be = 2000
    nb = pg.shape[0] // be
    return pl.pallas_call(
        _edge_body,
        grid=(nb,),
        in_specs=[
            pl.BlockSpec((be, _D), lambda i: (i, 0)),
            pl.BlockSpec((be, _D), lambda i: (i, 0)),
            pl.BlockSpec((be, _D_EDGE), lambda i: (i, 0)),
            pl.BlockSpec((_D_EDGE, 2 * _D), lambda i: (0, 0)),
            pl.BlockSpec((1, 2 * _D), lambda i: (0, 0)),
        ],
        out_specs=pl.BlockSpec((be, _D), lambda i: (i, 0)),
        out_shape=jax.ShapeDtypeStruct((pg.shape[0], _D), jnp.float32),
    )(pg, qg, ea, we, bcat)


def _tc_final(pa, pb, x, batch2d, wlt, bl2d):
    return pl.pallas_call(
        _final_body,
        out_shape=jax.ShapeDtypeStruct((_NUM_GRAPHS, _NUM_CLASS), jnp.float32),
    )(pa, pb, x, batch2d, wlt, bl2d)


def _sc_gather(p, q, dst, src):
    n_e = dst.shape[0]
    epw = n_e // _NW
    nchunk = epw // _CG
    mesh = plsc.VectorSubcoreMesh(core_axis_name="c", subcore_axis_name="s")

    @functools.partial(
        pl.kernel,
        mesh=mesh,
        out_type=[jax.ShapeDtypeStruct((n_e, _D), jnp.int32),
                  jax.ShapeDtypeStruct((n_e, _D), jnp.int32)],
        scratch_types=[
            pltpu.VMEM((_CG,), jnp.int32),
            pltpu.VMEM((_CG,), jnp.int32),
            pltpu.VMEM((_CG,), jnp.int32),
            pltpu.VMEM((_CG,), jnp.int32),
            pltpu.VMEM((2, _CG, _D), jnp.int32),
            pltpu.VMEM((2, _CG, _D), jnp.int32),
        ] + [pltpu.SemaphoreType.DMA] * 4,
    )
    def k(p_hbm, q_hbm, dst_hbm, src_hbm, pg_hbm, qg_hbm,
          idxd0, idxd1, idxs0, idxs1, bufp, bufq, gsem0, gsem1, wsem0, wsem1):
        wid = lax.axis_index("s") * _NC + lax.axis_index("c")
        idxd = (idxd0, idxd1)
        idxs = (idxs0, idxs1)
        gsems = (gsem0, gsem1)
        wsems = (wsem0, wsem1)

        def start(i, b):
            base = wid * epw + i * _CG
            pltpu.sync_copy(dst_hbm.at[pl.ds(base, _CG)], idxd[b])
            pltpu.sync_copy(src_hbm.at[pl.ds(base, _CG)], idxs[b])
            pltpu.async_copy(p_hbm.at[idxd[b]], bufp.at[b], gsems[b])
            pltpu.async_copy(q_hbm.at[idxs[b]], bufq.at[b], gsems[b])

        def wait_gather(b):
            pltpu.make_async_copy(p_hbm.at[idxd[b]], bufp.at[b], gsems[b]).wait()
            pltpu.make_async_copy(q_hbm.at[idxs[b]], bufq.at[b], gsems[b]).wait()

        def put(i, b):
            base = wid * epw + i * _CG
            pltpu.async_copy(bufp.at[b], pg_hbm.at[pl.ds(base, _CG)], wsems[b])
            pltpu.async_copy(bufq.at[b], qg_hbm.at[pl.ds(base, _CG)], wsems[b])

        def wait_put(i, b):
            base = wid * epw + i * _CG
            pltpu.make_async_copy(bufp.at[b], pg_hbm.at[pl.ds(base, _CG)], wsems[b]).wait()
            pltpu.make_async_copy(bufq.at[b], qg_hbm.at[pl.ds(base, _CG)], wsems[b]).wait()

        start(0, 0)
        start(1, 1)

        def body(j, carry):
            for b in range(2):
                i = 2 * j + b

                @pl.when(i < nchunk)
                def _():
                    wait_gather(b)
                    put(i, b)
                    wait_put(i, b)

                    @pl.when(i + 2 < nchunk)
                    def _():
                        start(i + 2, b)

            return carry

        lax.fori_loop(0, (nchunk + 1) // 2, body, 0)

    return k(p, q, dst, src)


def _sc_scatter(msg, dst):
    n_e = dst.shape[0]
    epw = n_e // _NW
    nchunk = epw // _CS
    mesh = plsc.VectorSubcoreMesh(core_axis_name="c", subcore_axis_name="s")

    @functools.partial(
        pl.kernel,
        mesh=mesh,
        out_type=jax.ShapeDtypeStruct((_NC, _NPAD, _D), jnp.float32),
        scratch_types=[
            pltpu.VMEM((_CS,), jnp.int32),
            pltpu.VMEM((_CS,), jnp.int32),
            pltpu.VMEM((2, _CS, _D), jnp.float32),
            pltpu.VMEM((_ZR, _D), jnp.float32),
            pltpu.VMEM_SHARED((_NPAD, _D), jnp.float32),
        ] + [pltpu.SemaphoreType.DMA] * 2,
    )
    def k(msg_hbm, dst_hbm, out_hbm, idx0, idx1, buf, zbuf, acc, lsem0, lsem1):
        cid = lax.axis_index("c")
        sid = lax.axis_index("s")
        wid = sid * _NC + cid
        idx = (idx0, idx1)
        lsems = (lsem0, lsem1)

        def zrow(r, carry):
            for kk in range(_D // 16):
                zbuf[r, pl.ds(kk * 16, 16)] = jnp.zeros((16,), jnp.float32)
            return carry

        lax.fori_loop(0, _ZR, zrow, 0)

        def zcopy(j, carry):
            pltpu.sync_copy(zbuf, acc.at[pl.ds(sid * _RPT + j * _ZR, _ZR)])
            return carry

        lax.fori_loop(0, _RPT // _ZR, zcopy, 0)
        plsc.subcore_barrier()

        def start(i, b):
            base = wid * epw + i * _CS
            pltpu.async_copy(dst_hbm.at[pl.ds(base, _CS)], idx[b], lsems[b])
            pltpu.async_copy(msg_hbm.at[pl.ds(base, _CS)], buf.at[b], lsems[b])

        def wait_load(i, b):
            base = wid * epw + i * _CS
            pltpu.make_async_copy(dst_hbm.at[pl.ds(base, _CS)], idx[b],
                                  lsems[b]).wait()
            pltpu.make_async_copy(msg_hbm.at[pl.ds(base, _CS)], buf.at[b],
                                  lsems[b]).wait()

        start(0, 0)
        start(1, 1)

        def body(j, carry):
            for b in range(2):
                i = 2 * j + b

                @pl.when(i < nchunk)
                def _():
                    wait_load(i, b)
                    pltpu.sync_copy(buf.at[b], acc.at[idx[b]], add=True)

                    @pl.when(i + 2 < nchunk)
                    def _():
                        start(i + 2, b)

            return carry

        lax.fori_loop(0, (nchunk + 1) // 2, body, 0)
        plsc.subcore_barrier()

        def drain(j, carry):
            r0 = sid * _RPT + j * _ZR
            pltpu.sync_copy(acc.at[pl.ds(r0, _ZR)], zbuf)
            pltpu.sync_copy(zbuf, out_hbm.at[cid, pl.ds(r0, _ZR)])
            return carry

        lax.fori_loop(0, _RPT // _ZR, drain, 0)

    return k(msg, dst)


def kernel(x, edge_index, edge_attr, batch, W_f, b_f, W_s, b_s, W_lin, b_lin):
    src = edge_index[0].astype(jnp.int32)
    dst = edge_index[1].astype(jnp.int32)

    wd = jnp.concatenate([W_f[:, :_D], W_s[:, :_D]], axis=0).T
    ws = jnp.concatenate([W_f[:, _D:2 * _D], W_s[:, _D:2 * _D]], axis=0).T
    we = jnp.concatenate([W_f[:, 2 * _D:], W_s[:, 2 * _D:]], axis=0).T
    bcat = jnp.concatenate([b_f, b_s]).reshape(1, 2 * _D)
    wlt = W_lin.T
    bl2d = b_lin.reshape(1, _NUM_CLASS)
    batch2d = batch.astype(jnp.int32).reshape(1, _N_NODES)

    p, q = _tc_pq(x, wd, ws)

    h = _N_EDGES // 2
    pga, qga = _sc_gather(p, q, dst[:h], src[:h])
    pgb, qgb = _sc_gather(p, q, dst[h:], src[h:])
    msga = _tc_edge(pga, qga, edge_attr[:h], we, bcat)
    msgb = _tc_edge(pgb, qgb, edge_attr[h:], we, bcat)
    pa = _sc_scatter(msga, dst[:h])
    pb = _sc_scatter(msgb, dst[h:])
    return _tc_final(pa, pb, x, batch2d, wlt, bl2d)


# speedup vs baseline: 1.1996x; 1.1048x over previous
import functools

import jax
import jax.numpy as jnp
from jax import lax
from jax.experimental import pallas as pl
from jax.experimental.pallas import tpu as pltpu
from jax.experimental.pallas import tpu_sc as plsc

_N_NODES = 10000
_N_EDGES = 320000
_D = 128
_D_EDGE = 16
_NUM_GRAPHS = 64
_NUM_CLASS = 10

_NC = 2
_NS = 16
_NW = _NC * _NS
_EPW = _N_EDGES // _NW
_CG = 200
_CS = 40
_NPAD = 10240
_ZR = 80
_RPT = _NPAD // _NS


_MASK_HI = -65536
_HALF = 0x8000


def _pack2(hi_f32, lo_f32):
    bh = lax.bitcast_convert_type(hi_f32, jnp.int32)
    bl = lax.bitcast_convert_type(lo_f32, jnp.int32)
    h = (bh + _HALF) & _MASK_HI
    l = lax.shift_right_logical(bl + _HALF, 16)
    return h | l


def _unpack_hi(p_i32):
    return lax.bitcast_convert_type(p_i32 & _MASK_HI, jnp.float32)


def _unpack_lo(p_i32):
    return lax.bitcast_convert_type(lax.shift_left(p_i32, 16), jnp.float32)


def _pq_body(x_ref, wd_ref, ws_ref, out_ref):
    xb = x_ref[...]
    pf = jnp.dot(xb, wd_ref[...], preferred_element_type=jnp.float32)
    qf = jnp.dot(xb, ws_ref[...], preferred_element_type=jnp.float32)
    out_ref[1] = _pack2(pf[:, :_D], pf[:, _D:])
    out_ref[0] = _pack2(qf[:, :_D], qf[:, _D:])


def _edge_body(pg_ref, qg_ref, ea_ref, we_ref, b_ref, out_ref):
    pg = pg_ref[0]
    qg = qg_ref[0]
    e = (jnp.dot(ea_ref[...], we_ref[...], preferred_element_type=jnp.float32)
         + b_ref[...])
    tf = _unpack_hi(pg) + _unpack_hi(qg) + e[:, :_D]
    ts = _unpack_lo(pg) + _unpack_lo(qg) + e[:, _D:]
    gate = 1.0 / (1.0 + jnp.exp(-tf))
    core = jnp.maximum(ts, 0.0) + jnp.log(1.0 + jnp.exp(-jnp.abs(ts)))
    out_ref[...] = gate * core


def _final_body(pa_ref, pb_ref, x_ref, batch_ref, wl_ref, bl_ref, out_ref):
    agg = (pa_ref[0, :_N_NODES, :] + pa_ref[1, :_N_NODES, :]
           + pb_ref[0, :_N_NODES, :] + pb_ref[1, :_N_NODES, :])
    out = jnp.maximum(agg + x_ref[...], 0.0)
    gids = lax.broadcasted_iota(jnp.int32, (_NUM_GRAPHS, _N_NODES), 0)
    m = (batch_ref[...] == gids).astype(jnp.float32)
    sums = jnp.dot(m, out, preferred_element_type=jnp.float32)
    counts = jnp.sum(m, axis=1, keepdims=True)
    pooled = sums / jnp.maximum(counts, 1.0)
    out_ref[...] = (jnp.dot(pooled, wl_ref[...],
                            preferred_element_type=jnp.float32) + bl_ref[...])


def _tc_pq(xpad, wd, ws):
    nb = 5
    rows = _NPAD // nb
    return pl.pallas_call(
        _pq_body,
        grid=(nb,),
        in_specs=[
            pl.BlockSpec((rows, _D), lambda i: (i, 0)),
            pl.BlockSpec((_D, 2 * _D), lambda i: (0, 0)),
            pl.BlockSpec((_D, 2 * _D), lambda i: (0, 0)),
        ],
        out_specs=pl.BlockSpec((2, rows, _D), lambda i: (0, i, 0)),
        out_shape=jax.ShapeDtypeStruct((2, _NPAD, _D), jnp.int32),
    )(xpad, wd, ws)


def _tc_edge(gat, ea, we, bcat):
    be = 2000
    nb = gat.shape[1] // be
    return pl.pallas_call(
        _edge_body,
        grid=(nb,),
        in_specs=[
            pl.BlockSpec((1, be, _D), lambda i: (1, i, 0)),
            pl.BlockSpec((1, be, _D), lambda i: (0, i, 0)),
            pl.BlockSpec((be, _D_EDGE), lambda i: (i, 0)),
            pl.BlockSpec((_D_EDGE, 2 * _D), lambda i: (0, 0)),
            pl.BlockSpec((1, 2 * _D), lambda i: (0, 0)),
        ],
        out_specs=pl.BlockSpec((be, _D), lambda i: (i, 0)),
        out_shape=jax.ShapeDtypeStruct((gat.shape[1], _D), jnp.float32),
    )(gat, gat, ea, we, bcat)


def _tc_final(pa, pb, x, batch2d, wlt, bl2d):
    return pl.pallas_call(
        _final_body,
        out_shape=jax.ShapeDtypeStruct((_NUM_GRAPHS, _NUM_CLASS), jnp.float32),
    )(pa, pb, x, batch2d, wlt, bl2d)


_CGS = 80
_TSR = 128


def _sc_gather(tbl, eidx):
    n_e = eidx.shape[0] // 2
    epw = n_e // _NS
    nchunk = epw // _CGS
    mesh = plsc.VectorSubcoreMesh(core_axis_name="c", subcore_axis_name="s")

    @functools.partial(
        pl.kernel,
        mesh=mesh,
        out_type=jax.ShapeDtypeStruct((2, n_e, _D), jnp.int32),
        scratch_types=[
            pltpu.VMEM((_CGS,), jnp.int32),
            pltpu.VMEM((_CGS,), jnp.int32),
            pltpu.VMEM((2, _CGS, _D), jnp.int32),
            pltpu.VMEM((_TSR, _D), jnp.int32),
            pltpu.VMEM_SHARED((_NPAD, _D), jnp.int32),
        ] + [pltpu.SemaphoreType.DMA] * 4,
    )
    def k(tbl_hbm, eidx_hbm, out_hbm, idx0, idx1, buf, stage, table,
          gsem0, gsem1, wsem0, wsem1):
        cid = lax.axis_index("c")
        sid = lax.axis_index("s")
        idx = (idx0, idx1)
        gsems = (gsem0, gsem1)
        wsems = (wsem0, wsem1)

        def stage_body(j, carry):
            r0 = sid * _RPT + j * _TSR
            pltpu.sync_copy(tbl_hbm.at[cid, pl.ds(r0, _TSR)], stage)
            pltpu.sync_copy(stage, table.at[pl.ds(r0, _TSR)])
            return carry

        lax.fori_loop(0, _RPT // _TSR, stage_body, 0)
        plsc.subcore_barrier()

        def start(i, b):
            base = sid * epw + i * _CGS
            pltpu.sync_copy(eidx_hbm.at[pl.ds(cid * n_e + base, _CGS)], idx[b])
            pltpu.async_copy(table.at[idx[b]], buf.at[b], gsems[b])

        def wait_gather(b):
            pltpu.make_async_copy(table.at[idx[b]], buf.at[b], gsems[b]).wait()

        def put(i, b):
            base = sid * epw + i * _CGS
            pltpu.async_copy(buf.at[b], out_hbm.at[cid, pl.ds(base, _CGS)],
                             wsems[b])

        def wait_put(i, b):
            base = sid * epw + i * _CGS
            pltpu.make_async_copy(buf.at[b], out_hbm.at[cid, pl.ds(base, _CGS)],
                                  wsems[b]).wait()

        start(0, 0)
        start(1, 1)

        def body(j, carry):
            for b in range(2):
                i = 2 * j + b

                @pl.when(i < nchunk)
                def _():
                    wait_gather(b)
                    put(i, b)
                    wait_put(i, b)

                    @pl.when(i + 2 < nchunk)
                    def _():
                        start(i + 2, b)

            return carry

        lax.fori_loop(0, (nchunk + 1) // 2, body, 0)

    return k(tbl, eidx)


def _sc_scatter(msg, dst):
    n_e = dst.shape[0]
    epw = n_e // _NW
    nchunk = epw // _CS
    mesh = plsc.VectorSubcoreMesh(core_axis_name="c", subcore_axis_name="s")

    @functools.partial(
        pl.kernel,
        mesh=mesh,
        out_type=jax.ShapeDtypeStruct((_NC, _NPAD, _D), jnp.float32),
        scratch_types=[
            pltpu.VMEM((_CS,), jnp.int32),
            pltpu.VMEM((_CS,), jnp.int32),
            pltpu.VMEM((2, _CS, _D), jnp.float32),
            pltpu.VMEM((_ZR, _D), jnp.float32),
            pltpu.VMEM_SHARED((_NPAD, _D), jnp.float32),
        ] + [pltpu.SemaphoreType.DMA] * 2,
    )
    def k(msg_hbm, dst_hbm, out_hbm, idx0, idx1, buf, zbuf, acc, lsem0, lsem1):
        cid = lax.axis_index("c")
        sid = lax.axis_index("s")
        wid = sid * _NC + cid
        idx = (idx0, idx1)
        lsems = (lsem0, lsem1)

        def zrow(r, carry):
            for kk in range(_D // 16):
                zbuf[r, pl.ds(kk * 16, 16)] = jnp.zeros((16,), jnp.float32)
            return carry

        lax.fori_loop(0, _ZR, zrow, 0)

        def zcopy(j, carry):
            pltpu.sync_copy(zbuf, acc.at[pl.ds(sid * _RPT + j * _ZR, _ZR)])
            return carry

        lax.fori_loop(0, _RPT // _ZR, zcopy, 0)
        plsc.subcore_barrier()

        def start(i, b):
            base = wid * epw + i * _CS
            pltpu.async_copy(dst_hbm.at[pl.ds(base, _CS)], idx[b], lsems[b])
            pltpu.async_copy(msg_hbm.at[pl.ds(base, _CS)], buf.at[b], lsems[b])

        def wait_load(i, b):
            base = wid * epw + i * _CS
            pltpu.make_async_copy(dst_hbm.at[pl.ds(base, _CS)], idx[b],
                                  lsems[b]).wait()
            pltpu.make_async_copy(msg_hbm.at[pl.ds(base, _CS)], buf.at[b],
                                  lsems[b]).wait()

        start(0, 0)
        start(1, 1)

        def body(j, carry):
            for b in range(2):
                i = 2 * j + b

                @pl.when(i < nchunk)
                def _():
                    wait_load(i, b)
                    pltpu.sync_copy(buf.at[b], acc.at[idx[b]], add=True)

                    @pl.when(i + 2 < nchunk)
                    def _():
                        start(i + 2, b)

            return carry

        lax.fori_loop(0, (nchunk + 1) // 2, body, 0)
        plsc.subcore_barrier()

        def drain(j, carry):
            r0 = sid * _RPT + j * _ZR
            pltpu.sync_copy(acc.at[pl.ds(r0, _ZR)], zbuf)
            pltpu.sync_copy(zbuf, out_hbm.at[cid, pl.ds(r0, _ZR)])
            return carry

        lax.fori_loop(0, _RPT // _ZR, drain, 0)

    return k(msg, dst)


def kernel(x, edge_index, edge_attr, batch, W_f, b_f, W_s, b_s, W_lin, b_lin):
    src = edge_index[0].astype(jnp.int32)
    dst = edge_index[1].astype(jnp.int32)

    wd = jnp.concatenate([W_f[:, :_D], W_s[:, :_D]], axis=0).T
    ws = jnp.concatenate([W_f[:, _D:2 * _D], W_s[:, _D:2 * _D]], axis=0).T
    we = jnp.concatenate([W_f[:, 2 * _D:], W_s[:, 2 * _D:]], axis=0).T
    bcat = jnp.concatenate([b_f, b_s]).reshape(1, 2 * _D)
    wlt = W_lin.T
    bl2d = b_lin.reshape(1, _NUM_CLASS)
    batch2d = batch.astype(jnp.int32).reshape(1, _N_NODES)

    xpad = jnp.concatenate(
        [x, jnp.zeros((_NPAD - _N_NODES, _D), jnp.float32)], axis=0)
    tbl = _tc_pq(xpad, wd, ws)
    ei32 = edge_index.astype(jnp.int32)

    h = _N_EDGES // 2
    gat_a = _sc_gather(tbl, ei32[:, :h].reshape(2 * h))
    gat_b = _sc_gather(tbl, ei32[:, h:].reshape(2 * h))
    msga = _tc_edge(gat_a, edge_attr[:h], we, bcat)
    msgb = _tc_edge(gat_b, edge_attr[h:], we, bcat)
    pa = _sc_scatter(msga, dst[:h])
    pb = _sc_scatter(msgb, dst[h:])
    return _tc_final(pa, pb, x, batch2d, wlt, bl2d)
